# baseline (speedup 1.0000x reference)
import functools

import jax
import jax.numpy as jnp
from jax import lax
from jax.experimental import pallas as pl
from jax.experimental.pallas import tpu as pltpu
from jax.experimental.pallas import tpu_sc as plsc

BATCH = 16384
D = 32
NC = 2
NS = 16
NW = NC * NS
BPW = BATCH // NW

_mesh = plsc.VectorSubcoreMesh(core_axis_name="c", subcore_axis_name="s")


@functools.partial(
    pl.kernel,
    out_type=jax.ShapeDtypeStruct((BATCH,), jnp.float32),
    mesh=_mesh,
    scratch_types=[
        pltpu.VMEM((BPW,), jnp.int32),
        pltpu.VMEM((BPW,), jnp.int32),
        pltpu.VMEM((BPW, D), jnp.float32),
        pltpu.VMEM((BPW, D), jnp.float32),
        pltpu.VMEM((48,), jnp.float32),
        pltpu.VMEM((BPW,), jnp.float32),
        pltpu.SemaphoreType.DMA,
    ],
    compiler_params=pltpu.CompilerParams(
        needs_layout_passes=False, use_tc_tiling_on_sc=False),
)
def _gmf_sc(uid_hbm, iid_hbm, ut_hbm, it_hbm, wb_hbm, out_hbm,
            uidx, iidx, urows, irows, wv, outv, sem):
    wid = lax.axis_index("s") * NC + lax.axis_index("c")
    base = wid * BPW

    pltpu.sync_copy(uid_hbm.at[pl.ds(base, BPW)], uidx)
    pltpu.sync_copy(iid_hbm.at[pl.ds(base, BPW)], iidx)
    cu = pltpu.async_copy(ut_hbm.at[uidx], urows, sem)
    ci = pltpu.async_copy(it_hbm.at[iidx], irows, sem)
    pltpu.sync_copy(wb_hbm, wv)
    cu.wait()
    ci.wait()

    w_lo = wv[pl.ds(0, 16)]
    w_hi = wv[pl.ds(16, 16)]
    b0 = wv[pl.ds(32, 16)][0]

    lanes = lax.iota(jnp.int32, 16)

    def group_body(g, _):
        base_r = g * 16
        acc = jnp.zeros((16,), jnp.float32)
        for j in range(16):
            r = base_r + j
            u_pl = urows[r, pl.ds(0, 16)] * irows[r, pl.ds(0, 16)] * w_lo
            u_ph = urows[r, pl.ds(16, 16)] * irows[r, pl.ds(16, 16)] * w_hi
            s = jnp.sum(u_pl + u_ph)
            acc = jnp.where(lanes == j, s, acc)
        v = 1.0 / (1.0 + jnp.exp(-(acc + b0)))
        off = pl.multiple_of(g * 16, 16)
        outv[pl.ds(off, 16)] = v
        return 0

    lax.fori_loop(0, BPW // 16, group_body, 0)

    pltpu.sync_copy(outv, out_hbm.at[pl.ds(base, BPW)])


def kernel(user_ids, item_ids, user_table, item_table, W, b):
    wb = jnp.zeros((48,), jnp.float32)
    wb = wb.at[:D].set(W.reshape(D)).at[D].set(b[0])
    return _gmf_sc(user_ids.astype(jnp.int32), item_ids.astype(jnp.int32),
                   user_table, item_table, wb)
